# baseline (speedup 1.0000x reference)
import functools

import jax
import jax.numpy as jnp
from jax import lax
from jax.experimental import pallas as pl
from jax.experimental.pallas import tpu as pltpu
from jax.experimental.pallas import tpu_sc as plsc

B, C, H, W, K = 16, 80, 128, 128, 128
HW = H * W
HM_W, WH_W, OFF_W = 1.0, 0.1, 1.0


_ROWS = B * C
_BLK = 64


def _focal_block(x_ref, g_ref, out_ref):
    @pl.when(pl.program_id(0) == 0)
    def _init():
        out_ref[0] = 0.0
        out_ref[1] = 0.0
        out_ref[2] = 0.0

    x = x_ref[...]
    g = g_ref[...]
    pred = jnp.clip(1.0 / (1.0 + jnp.exp(-x)), 1e-4, 1.0 - 1e-4)
    pos = (g == 1.0).astype(jnp.float32)
    neg = (g < 1.0).astype(jnp.float32)
    one_m_pred = 1.0 - pred
    gm = 1.0 - g
    neg_w = (gm * gm) * (gm * gm)
    pos_l = jnp.log(pred) * (one_m_pred * one_m_pred) * pos
    neg_l = jnp.log(one_m_pred) * (pred * pred) * neg_w * neg
    out_ref[0] += jnp.sum(pos_l)
    out_ref[1] += jnp.sum(neg_l)
    out_ref[2] += jnp.sum(pos)


_focal = pl.pallas_call(
    _focal_block,
    grid=(_ROWS // _BLK,),
    in_specs=[
        pl.BlockSpec((_BLK, HW), lambda i: (i, 0)),
        pl.BlockSpec((_BLK, HW), lambda i: (i, 0)),
    ],
    out_specs=pl.BlockSpec(memory_space=pltpu.SMEM),
    out_shape=jax.ShapeDtypeStruct((3,), jnp.float32),
)


_NC, _NS = 2, 16
_NW = _NC * _NS
_NPAIR = B * K
_PPW = _NPAIR // _NW

_sc_mesh = plsc.VectorSubcoreMesh(core_axis_name="c", subcore_axis_name="s")


@functools.partial(
    pl.kernel,
    mesh=_sc_mesh,
    out_type=[
        jax.ShapeDtypeStruct((_NW, 16), jnp.float32),
        jax.ShapeDtypeStruct((_NW, 16), jnp.float32),
        jax.ShapeDtypeStruct((_NW, 16), jnp.float32),
    ],
    scratch_types=[
        pltpu.VMEM((_PPW,), jnp.int32),
        pltpu.VMEM((2 * _PPW,), jnp.int32),
        pltpu.VMEM((2 * _PPW,), jnp.float32),
        pltpu.VMEM((2 * _PPW,), jnp.float32),
        pltpu.VMEM((_PPW,), jnp.float32),
        pltpu.VMEM((_PPW,), jnp.float32),
        pltpu.VMEM((_PPW,), jnp.float32),
        pltpu.VMEM((_PPW,), jnp.float32),
        pltpu.VMEM((_PPW,), jnp.float32),
        pltpu.VMEM((16,), jnp.float32),
        pltpu.SemaphoreType.DMA,
    ],
)
def _sc_reg(wh_hbm, reg_hbm, whgt_hbm, reggt_hbm, mask_hbm, ind_hbm,
            out_wh, out_off, out_m,
            ind_v, idx_v, whv, regv, gt0, gt1, gt2, gt3, mv, accv, sem):
    wid = lax.axis_index("s") * _NC + lax.axis_index("c")
    base = wid * _PPW
    b = wid // 2
    ko = (wid % 2) * _PPW

    pltpu.sync_copy(ind_hbm.at[pl.ds(base, _PPW)], ind_v)
    pltpu.sync_copy(mask_hbm.at[pl.ds(base, _PPW)], mv)
    gtbase = b * (2 * K) + ko
    pltpu.sync_copy(whgt_hbm.at[pl.ds(gtbase, _PPW)], gt0)
    pltpu.sync_copy(whgt_hbm.at[pl.ds(gtbase + K, _PPW)], gt1)
    pltpu.sync_copy(reggt_hbm.at[pl.ds(gtbase, _PPW)], gt2)
    pltpu.sync_copy(reggt_hbm.at[pl.ds(gtbase + K, _PPW)], gt3)

    chan = b * (2 * HW)
    for j in range(_PPW // 16):
        v = ind_v[pl.ds(j * 16, 16)]
        idx_v[pl.ds(j * 16, 16)] = v + chan
        idx_v[pl.ds(_PPW + j * 16, 16)] = v + (chan + HW)

    pltpu.async_copy(wh_hbm.at[idx_v], whv, sem).wait()
    pltpu.async_copy(reg_hbm.at[idx_v], regv, sem).wait()

    acc_wh = jnp.zeros((16,), jnp.float32)
    acc_off = jnp.zeros((16,), jnp.float32)
    acc_m = jnp.zeros((16,), jnp.float32)
    for j in range(_PPW // 16):
        sl = pl.ds(j * 16, 16)
        sl1 = pl.ds(_PPW + j * 16, 16)
        m = mv[sl]
        acc_m = acc_m + m
        acc_wh = acc_wh + jnp.abs(whv[sl] * m - gt0[sl] * m) \
                        + jnp.abs(whv[sl1] * m - gt1[sl] * m)
        acc_off = acc_off + jnp.abs(regv[sl] * m - gt2[sl] * m) \
                          + jnp.abs(regv[sl1] * m - gt3[sl] * m)

    accv[...] = acc_wh
    pltpu.sync_copy(accv, out_wh.at[wid])
    accv[...] = acc_off
    pltpu.sync_copy(accv, out_off.at[wid])
    accv[...] = acc_m
    pltpu.sync_copy(accv, out_m.at[wid])


def kernel(hm_out, hm_gt, wh_out, wh_gt, reg_out, reg_gt, reg_mask, ind):
    sums = _focal(hm_out.reshape(_ROWS, HW), hm_gt.reshape(_ROWS, HW))
    pos_sum, neg_sum, num_pos = sums[0], sums[1], sums[2]

    wh_p, off_p, m_p = _sc_reg(
        wh_out.reshape(-1),
        reg_out.reshape(-1),
        jnp.transpose(wh_gt, (0, 2, 1)).reshape(-1),
        jnp.transpose(reg_gt, (0, 2, 1)).reshape(-1),
        reg_mask.reshape(-1),
        ind.reshape(-1),
    )

    denom = jnp.maximum(num_pos, 1.0)
    hm_loss = jnp.where(num_pos == 0, -neg_sum, -(pos_sum + neg_sum) / denom)
    msum = 2.0 * jnp.sum(m_p) + 0.0001
    wh_loss = jnp.sum(wh_p) / msum
    off_loss = jnp.sum(off_p) / msum
    loss = HM_W * hm_loss + WH_W * wh_loss + OFF_W * off_loss
    return (loss, hm_loss, wh_loss, off_loss)
